# baseline (speedup 1.0000x reference)
import functools

import jax
import jax.numpy as jnp
from jax import lax
from jax.experimental import pallas as pl
from jax.experimental.pallas import tpu as pltpu
from jax.experimental.pallas import tpu_sc as plsc

B = 16384
D = 64
DP = D // 2
NC = 2
NS = 16
NW = NC * NS
BPW = B // NW
NCHUNK = 4
CHUNK = BPW // NCHUNK
L = 16
NGRP = BPW // L


def _gmf_body(users_ref, items_ref, ut_ref, it_ref, wb_ref, out_ref,
              idx_u, idx_i, rows_u, rows_i, out_v, wb_v, sem_u, sem_i):
    c = lax.axis_index("c")
    s = lax.axis_index("s")
    wid = s * NC + c
    base = wid * BPW

    pltpu.sync_copy(users_ref.at[wid], idx_u)
    pltpu.sync_copy(items_ref.at[wid], idx_i)
    pltpu.sync_copy(wb_ref, wb_v)

    copies = []
    for j in range(NCHUNK):
        copies.append(pltpu.async_copy(
            ut_ref.at[idx_u.at[j]], rows_u.at[pl.ds(j * CHUNK, CHUNK)], sem_u))
        copies.append(pltpu.async_copy(
            it_ref.at[idx_i.at[j]], rows_i.at[pl.ds(j * CHUNK, CHUNK)], sem_i))
    for cp in copies:
        cp.wait()

    wvecs = [wb_v[pl.ds(k * L, L)] for k in range(D // L)]
    wscal = [wvecs[d // L][d % L] for d in range(D)]
    bias_vec = wb_v[pl.ds(D, L)]
    iota = lax.iota(jnp.int32, L)
    cols = [jnp.full((L,), cc, jnp.int32) for cc in range(DP)]
    himask = jnp.full((L,), -65536, jnp.int32)

    def group_body(g, carry):
        row_idx = g * L + iota
        acc = jnp.zeros((L,), jnp.float32)
        for cc in range(DP):
            xu = plsc.load_gather(rows_u, [row_idx, cols[cc]])
            xv = plsc.load_gather(rows_i, [row_idx, cols[cc]])
            ulo = plsc.bitcast(xu << 16, jnp.float32)
            vlo = plsc.bitcast(xv << 16, jnp.float32)
            uhi = plsc.bitcast(xu & himask, jnp.float32)
            vhi = plsc.bitcast(xv & himask, jnp.float32)
            acc = acc + (ulo * vlo) * wscal[2 * cc]
            acc = acc + (uhi * vhi) * wscal[2 * cc + 1]
        x = acc + bias_vec
        out_v[pl.ds(g * L, L)] = 1.0 / (1.0 + jnp.exp(-x))
        return carry

    lax.fori_loop(0, NGRP, group_body, 0)

    pltpu.sync_copy(out_v, out_ref.at[pl.ds(base, BPW)])


_gmf = functools.partial(
    pl.kernel,
    out_type=jax.ShapeDtypeStruct((B,), jnp.float32),
    mesh=plsc.VectorSubcoreMesh(core_axis_name="c", subcore_axis_name="s"),
    compiler_params=pltpu.CompilerParams(
        needs_layout_passes=False, use_tc_tiling_on_sc=False),
    scratch_types=[
        pltpu.VMEM((NCHUNK, CHUNK), jnp.int32),
        pltpu.VMEM((NCHUNK, CHUNK), jnp.int32),
        pltpu.VMEM((BPW, DP), jnp.int32),
        pltpu.VMEM((BPW, DP), jnp.int32),
        pltpu.VMEM((BPW,), jnp.float32),
        pltpu.VMEM((D + L,), jnp.float32),
        pltpu.SemaphoreType.DMA,
        pltpu.SemaphoreType.DMA,
    ],
)(_gmf_body)


def _pack_table(t):
    tb = t.astype(jnp.bfloat16).reshape(t.shape[0], DP, 2)
    return lax.bitcast_convert_type(tb, jnp.int32)


def kernel(users, items, user_table, item_table, predict_w, predict_b):
    users2 = users.astype(jnp.int32).reshape(NW, NCHUNK, CHUNK)
    items2 = items.astype(jnp.int32).reshape(NW, NCHUNK, CHUNK)
    wb = jnp.concatenate(
        [predict_w.reshape(-1), jnp.full((L,), predict_b[0], jnp.float32)])
    return _gmf(users2, items2, _pack_table(user_table),
                _pack_table(item_table), wb)


# speedup vs baseline: 2.8388x; 2.8388x over previous
import functools

import jax
import jax.numpy as jnp
from jax import lax
from jax.experimental import pallas as pl
from jax.experimental.pallas import tpu as pltpu
from jax.experimental.pallas import tpu_sc as plsc

B = 16384
D = 64
VP = 500000
NC = 2
NS = 16
NW = NC * NS
BPW = B // NW
NCHUNK = 4
CHUNK = BPW // NCHUNK
L = 16
GPC = CHUNK // L


def _gmf_body(users_ref, items_ref, ut_ref, it_ref, wb_ref, out_ref,
              idx_u, idx_i, pr_u, pr_i, bu, bi, out_v, wb_v, sem_u, sem_i):
    c = lax.axis_index("c")
    s = lax.axis_index("s")
    wid = s * NC + c
    base = wid * BPW

    pltpu.sync_copy(users_ref.at[pl.ds(base, BPW)], idx_u)
    pltpu.sync_copy(items_ref.at[pl.ds(base, BPW)], idx_i)
    pltpu.sync_copy(wb_ref, wb_v)

    def halve(k, carry):
        pr_u[pl.ds(k * L, L)] = idx_u[pl.ds(k * L, L)] >> 1
        pr_i[pl.ds(k * L, L)] = idx_i[pl.ds(k * L, L)] >> 1
        return carry

    lax.fori_loop(0, BPW // L, halve, 0)

    def fire(j):
        b = j % 2
        cu = pltpu.async_copy(
            ut_ref.at[pr_u.at[pl.ds(j * CHUNK, CHUNK)]], bu.at[b], sem_u)
        ci = pltpu.async_copy(
            it_ref.at[pr_i.at[pl.ds(j * CHUNK, CHUNK)]], bi.at[b], sem_i)
        return cu, ci

    wvecs = [wb_v[pl.ds(k * L, L)] for k in range(D // L)]
    wscal = [wvecs[d // L][d % L] for d in range(D)]
    bias_vec = wb_v[pl.ds(D, L)]
    iota = lax.iota(jnp.int32, L)
    sixty4 = jnp.full((L,), 64, jnp.int32)

    inflight = fire(0)
    for j in range(NCHUNK):
        nxt = fire(j + 1) if j + 1 < NCHUNK else None
        inflight[0].wait()
        inflight[1].wait()
        b = j % 2
        bu_j = bu.at[b]
        bi_j = bi.at[b]
        for g in range(GPC):
            row16 = iota + (g * L)
            paru = (idx_u[pl.ds(j * CHUNK + g * L, L)] & 1) * sixty4
            pari = (idx_i[pl.ds(j * CHUNK + g * L, L)] & 1) * sixty4
            acc = jnp.zeros((L,), jnp.float32)
            for cc in range(D):
                u = plsc.load_gather(bu_j, [row16, paru + cc])
                v = plsc.load_gather(bi_j, [row16, pari + cc])
                acc = acc + (u * v) * wscal[cc]
            x = acc + bias_vec
            out_v[pl.ds(j * CHUNK + g * L, L)] = 1.0 / (1.0 + jnp.exp(-x))
        inflight = nxt

    pltpu.sync_copy(out_v, out_ref.at[pl.ds(base, BPW)])


_gmf = functools.partial(
    pl.kernel,
    out_type=jax.ShapeDtypeStruct((B,), jnp.float32),
    mesh=plsc.VectorSubcoreMesh(core_axis_name="c", subcore_axis_name="s"),
    compiler_params=pltpu.CompilerParams(
        needs_layout_passes=False, use_tc_tiling_on_sc=True),
    scratch_types=[
        pltpu.VMEM((BPW,), jnp.int32),
        pltpu.VMEM((BPW,), jnp.int32),
        pltpu.VMEM((BPW,), jnp.int32),
        pltpu.VMEM((BPW,), jnp.int32),
        pltpu.VMEM((2, CHUNK, 2 * D), jnp.float32),
        pltpu.VMEM((2, CHUNK, 2 * D), jnp.float32),
        pltpu.VMEM((BPW,), jnp.float32),
        pltpu.VMEM((D + L,), jnp.float32),
        pltpu.SemaphoreType.DMA,
        pltpu.SemaphoreType.DMA,
    ],
)(_gmf_body)


def kernel(users, items, user_table, item_table, predict_w, predict_b):
    wb = jnp.concatenate(
        [predict_w.reshape(-1), jnp.full((L,), predict_b[0], jnp.float32)])
    return _gmf(users.astype(jnp.int32), items.astype(jnp.int32),
                user_table.reshape(VP, 2 * D), item_table.reshape(VP, 2 * D),
                wb)


# speedup vs baseline: 3.6452x; 1.2841x over previous
import functools

import jax
import jax.numpy as jnp
from jax import lax
from jax.experimental import pallas as pl
from jax.experimental.pallas import tpu as pltpu
from jax.experimental.pallas import tpu_sc as plsc

B = 16384
D = 64
V = 1000000
VP = V // 2
NC = 2
NS = 16
NW = NC * NS
BPW = B // NW
NCHUNK = 4
CHUNK = BPW // NCHUNK
L = 16
GPC = CHUNK // L

TBLK = 6400
TGRID = (V + TBLK - 1) // TBLK


def _transpose_body(inT_ref, out_ref):
    x = inT_ref[...]
    r = lax.broadcasted_iota(jnp.int32, (D, D), 0)
    c = lax.broadcasted_iota(jnp.int32, (D, D), 1)
    ident = (r == c).astype(jnp.float32)
    xt = lax.dot_general(x, ident, (((0,), (0,)), ((), ())),
                         preferred_element_type=jnp.float32)
    out_ref[:, 0:D] = xt
    out_ref[:, D:2 * D] = jnp.zeros((TBLK, D), jnp.float32)


_transpose = pl.pallas_call(
    _transpose_body,
    grid=(TGRID,),
    in_specs=[pl.BlockSpec((D, TBLK), lambda k: (0, k))],
    out_specs=pl.BlockSpec((TBLK, 2 * D), lambda k: (k, 0)),
    out_shape=jax.ShapeDtypeStruct((V, 2 * D), jnp.float32),
)


def _gmf_body(users_ref, items_ref, ut_ref, it_ref, wb_ref, out_ref,
              idx_u, idx_i, pr_i, bu, bi, out_v, wb_v, sem_u, sem_i):
    c = lax.axis_index("c")
    s = lax.axis_index("s")
    wid = s * NC + c
    base = wid * BPW

    pltpu.sync_copy(users_ref.at[pl.ds(base, BPW)], idx_u)
    pltpu.sync_copy(items_ref.at[pl.ds(base, BPW)], idx_i)
    pltpu.sync_copy(wb_ref, wb_v)

    def halve(k, carry):
        pr_i[pl.ds(k * L, L)] = idx_i[pl.ds(k * L, L)] >> 1
        return carry

    lax.fori_loop(0, BPW // L, halve, 0)

    def fire(j):
        b = j % 2
        cu = pltpu.async_copy(
            ut_ref.at[idx_u.at[pl.ds(j * CHUNK, CHUNK)]], bu.at[b], sem_u)
        ci = pltpu.async_copy(
            it_ref.at[pr_i.at[pl.ds(j * CHUNK, CHUNK)]], bi.at[b], sem_i)
        return cu, ci

    wvecs = [wb_v[pl.ds(k * L, L)] for k in range(D // L)]
    wscal = [wvecs[d // L][d % L] for d in range(D)]
    bias_vec = wb_v[pl.ds(D, L)]
    iota = lax.iota(jnp.int32, L)
    sixty4 = jnp.full((L,), D, jnp.int32)

    inflight = fire(0)
    for j in range(NCHUNK):
        nxt = fire(j + 1) if j + 1 < NCHUNK else None
        inflight[0].wait()
        inflight[1].wait()
        b = j % 2
        bu_j = bu.at[b]
        bi_j = bi.at[b]
        for g in range(GPC):
            row16 = iota + (g * L)
            pari = (idx_i[pl.ds(j * CHUNK + g * L, L)] & 1) * sixty4
            acc = jnp.zeros((L,), jnp.float32)
            for cc in range(D):
                u = plsc.load_gather(bu_j, [row16, jnp.full((L,), cc, jnp.int32)])
                v = plsc.load_gather(bi_j, [row16, pari + cc])
                acc = acc + (u * v) * wscal[cc]
            x = acc + bias_vec
            out_v[pl.ds(j * CHUNK + g * L, L)] = 1.0 / (1.0 + jnp.exp(-x))
        inflight = nxt

    pltpu.sync_copy(out_v, out_ref.at[pl.ds(base, BPW)])


_gmf = functools.partial(
    pl.kernel,
    out_type=jax.ShapeDtypeStruct((B,), jnp.float32),
    mesh=plsc.VectorSubcoreMesh(core_axis_name="c", subcore_axis_name="s"),
    compiler_params=pltpu.CompilerParams(
        needs_layout_passes=False, use_tc_tiling_on_sc=True),
    scratch_types=[
        pltpu.VMEM((BPW,), jnp.int32),
        pltpu.VMEM((BPW,), jnp.int32),
        pltpu.VMEM((BPW,), jnp.int32),
        pltpu.VMEM((2, CHUNK, 2 * D), jnp.float32),
        pltpu.VMEM((2, CHUNK, 2 * D), jnp.float32),
        pltpu.VMEM((BPW,), jnp.float32),
        pltpu.VMEM((D + L,), jnp.float32),
        pltpu.SemaphoreType.DMA,
        pltpu.SemaphoreType.DMA,
    ],
)(_gmf_body)


def kernel(users, items, user_table, item_table, predict_w, predict_b):
    wb = jnp.concatenate(
        [predict_w.reshape(-1), jnp.full((L,), predict_b[0], jnp.float32)])
    ut_rows = _transpose(user_table.T)
    return _gmf(users.astype(jnp.int32), items.astype(jnp.int32),
                ut_rows, item_table.reshape(VP, 2 * D), wb)
